# baseline (speedup 1.0000x reference)
import functools

import jax
import jax.numpy as jnp
from jax import lax
from jax.experimental import pallas as pl
from jax.experimental.pallas import tpu as pltpu
from jax.experimental.pallas import tpu_sc as plsc

N_CODES = 1024
CODE_DIM = 64
K_TOP = 8
INV_TEMP = 10.0

NW = 32
CHUNK = 96


def _topk_kernel(x_ref, wt_ref, b_ref, e_ref, iota_ref,
                 z_ref, wn_ref, idx8_ref, idx_ref, *, nsteps):
    r = x_ref.shape[0]
    x = x_ref[...]
    z = lax.dot_general(x, wt_ref[...], (((1,), (0,)), ((), ())),
                        preferred_element_type=jnp.float32,
                        precision=lax.Precision.DEFAULT) + b_ref[...]
    z_ref[...] = z
    e = e_ref[...]
    e2 = jnp.sum(e * e, axis=1)
    z2 = jnp.sum(z * z, axis=1, keepdims=True)
    cross = lax.dot_general(-2.0 * z, e, (((1,), (1,)), ((), ())),
                            preferred_element_type=jnp.float32,
                            precision=lax.Precision.DEFAULT)
    sq = (z2 + e2[None, :]) + cross

    BIG = jnp.float32(3e38)
    lanes = N_CODES // K_TOP
    c = [sq[:, g * lanes:(g + 1) * lanes] for g in range(K_TOP)]
    _NET = [(0, 1), (2, 3), (4, 5), (6, 7),
            (0, 2), (1, 3), (4, 6), (5, 7),
            (1, 2), (5, 6),
            (0, 4), (1, 5), (2, 6), (3, 7),
            (2, 4), (3, 5),
            (1, 2), (3, 4), (5, 6)]
    for lo_i, hi_i in _NET:
        lo = jnp.minimum(c[lo_i], c[hi_i])
        hi = jnp.maximum(c[lo_i], c[hi_i])
        c[lo_i], c[hi_i] = lo, hi
    vals = []
    idxs = []
    for j in range(K_TOP):
        m = jnp.min(c[0], axis=1, keepdims=True)
        vals.append(m)
        mask = jnp.where(sq == m, 1.0, 0.0)
        idxf = lax.dot_general(mask, iota_ref[...],
                               (((1,), (0,)), ((), ())),
                               preferred_element_type=jnp.float32)
        idxs.append(idxf)
        if j < K_TOP - 1:
            eq = c[0] == m
            depth = K_TOP - j
            for lvl in range(depth - 1):
                c[lvl] = jnp.where(eq, c[lvl + 1], c[lvl])
            c[depth - 1] = jnp.where(eq, BIG, c[depth - 1])

    m_stack = jnp.concatenate(vals, axis=1)
    d_vals = jnp.sqrt(jnp.clip(m_stack, 1e-12, None))
    d0 = d_vals[:, 0:1]
    wexp = jnp.exp((d0 - d_vals) * INV_TEMP)
    denom = jnp.sum(wexp, axis=1, keepdims=True)
    wn_ref[...] = wexp / denom
    idx8 = jnp.concatenate(idxs, axis=1)
    idx8 = jnp.clip(idx8, 0.0, float(N_CODES - 1))
    idx8_ref[...] = idx8.astype(jnp.int32)
    idx_ref[...] = jnp.clip(idxs[0], 0.0, float(N_CODES - 1)).astype(jnp.int32)


def _dyn_gather(x, idx):
    dn = lax.GatherDimensionNumbers(offset_dims=(), collapsed_slice_dims=(0,),
                                    start_index_map=(0,))
    return lax.gather(x, idx[:, None], dn, slice_sizes=(1,),
                      mode=lax.GatherScatterMode.PROMISE_IN_BOUNDS)


def _sc_combine(idx_hbm, wn_hbm, z_hbm, e_hbm,
                q_hbm, usage_hbm, loss_hbm,
                e_v, idx_v, wn_v, z_v, q_v, usage2d, loss_acc):
    n = z_hbm.shape[0] // CODE_DIM
    rows_w = n // NW
    wid = lax.axis_index("s") * 2 + lax.axis_index("c")
    base = wid * rows_w

    pltpu.sync_copy(e_hbm, e_v)

    zero16 = jnp.zeros((16,), jnp.float32)
    loss_acc[...] = zero16

    def _zero_body(i, _):
        usage2d[pl.ds(16 * i, 16)] = zero16
        return ()
    lax.fori_loop(0, N_CODES, _zero_body, ())

    li = lax.iota(jnp.int32, 16)

    for t in range(rows_w // CHUNK):
        r0 = base + t * CHUNK
        pltpu.sync_copy(idx_hbm.at[pl.ds(r0 * K_TOP, CHUNK * K_TOP)], idx_v)
        pltpu.sync_copy(wn_hbm.at[pl.ds(r0 * K_TOP, CHUNK * K_TOP)], wn_v)
        pltpu.sync_copy(z_hbm.at[pl.ds(r0 * CODE_DIM, CHUNK * CODE_DIM)], z_v)

        def _pair_body(g, _):
            idx16 = idx_v[pl.ds(16 * g, 16)]
            w16 = wn_v[pl.ds(16 * g, 16)]
            uaddr = idx16 * 16 + li
            cur = plsc.load_gather(usage2d, [uaddr])
            plsc.store_scatter(usage2d, [uaddr], cur + w16)
            for half in range(2):
                row = 2 * g + half
                segs = []
                for cseg in range(4):
                    acc = zero16
                    for j in range(K_TOP):
                        sel = jnp.full((16,), half * K_TOP + j, jnp.int32)
                        code = _dyn_gather(idx16, sel)
                        w = _dyn_gather(w16, sel)
                        atom = plsc.load_gather(
                            e_v, [code * CODE_DIM + li + 16 * cseg])
                        acc = acc + w * atom
                    segs.append(acc)
                for cseg in range(4):
                    q_v[pl.ds(row * CODE_DIM + 16 * cseg, 16)] = segs[cseg]
                    d = z_v[pl.ds(row * CODE_DIM + 16 * cseg, 16)] - segs[cseg]
                    loss_acc[...] = loss_acc[...] + d * d
            return ()
        lax.fori_loop(0, CHUNK // 2, _pair_body, ())
        pltpu.sync_copy(q_v, q_hbm.at[pl.ds(r0 * CODE_DIM, CHUNK * CODE_DIM)])

    pltpu.sync_copy(usage2d, usage_hbm.at[wid])
    pltpu.sync_copy(loss_acc, loss_hbm.at[wid])


def _run_sc(idx8_flat, wn_flat, z_flat, e_flat, n):
    sc = functools.partial(
        pl.kernel,
        out_type=[
            jax.ShapeDtypeStruct((n * CODE_DIM,), jnp.float32),
            jax.ShapeDtypeStruct((NW, N_CODES * 16), jnp.float32),
            jax.ShapeDtypeStruct((NW, 16), jnp.float32),
        ],
        mesh=plsc.VectorSubcoreMesh(core_axis_name="c", subcore_axis_name="s"),
        scratch_types=[
            pltpu.VMEM((N_CODES * CODE_DIM,), jnp.float32),
            pltpu.VMEM((CHUNK * K_TOP,), jnp.int32),
            pltpu.VMEM((CHUNK * K_TOP,), jnp.float32),
            pltpu.VMEM((CHUNK * CODE_DIM,), jnp.float32),
            pltpu.VMEM((CHUNK * CODE_DIM,), jnp.float32),
            pltpu.VMEM((N_CODES * 16,), jnp.float32),
            pltpu.VMEM((16,), jnp.float32),
        ],
        compiler_params=pltpu.CompilerParams(needs_layout_passes=False),
    )(_sc_combine)
    return sc(idx8_flat, wn_flat, z_flat, e_flat)


def _finalize_kernel(u3_ref, lp_ref, loss_ref, ent_ref, *, n_rows):
    u = jnp.sum(u3_ref[...], axis=0)
    usage = jnp.sum(u, axis=1)
    avg = usage * jnp.float32(1.0 / n_rows)
    ent = -jnp.sum(avg * jnp.log(avg + 1e-8))
    ent_ref[...] = jnp.full((1, 1), ent, jnp.float32)
    loss = jnp.sum(lp_ref[...]) / jnp.float32(n_rows * CODE_DIM)
    loss_ref[...] = jnp.full((1, 1), loss, jnp.float32)


def kernel(slot_features, W, b_lin, embed):
    b, k, d_model = slot_features.shape
    n = b * k
    x = slot_features.reshape(n, d_model)
    wt = W.T
    bb = b_lin.reshape(1, CODE_DIM)

    r = 512
    while n % r:
        r //= 2
    nsteps = n // r

    z_flat, wn, idx8, idx = pl.pallas_call(
        functools.partial(_topk_kernel, nsteps=nsteps),
        grid=(nsteps,),
        in_specs=[
            pl.BlockSpec((r, d_model), lambda i: (i, 0)),
            pl.BlockSpec((d_model, CODE_DIM), lambda i: (0, 0)),
            pl.BlockSpec((1, CODE_DIM), lambda i: (0, 0)),
            pl.BlockSpec((N_CODES, CODE_DIM), lambda i: (0, 0)),
            pl.BlockSpec((N_CODES, 1), lambda i: (0, 0)),
        ],
        out_specs=[
            pl.BlockSpec((r, CODE_DIM), lambda i: (i, 0)),
            pl.BlockSpec((r, K_TOP), lambda i: (i, 0)),
            pl.BlockSpec((r, K_TOP), lambda i: (i, 0)),
            pl.BlockSpec((r, 1), lambda i: (i, 0)),
        ],
        out_shape=[
            jax.ShapeDtypeStruct((n, CODE_DIM), jnp.float32),
            jax.ShapeDtypeStruct((n, K_TOP), jnp.float32),
            jax.ShapeDtypeStruct((n, K_TOP), jnp.int32),
            jax.ShapeDtypeStruct((n, 1), jnp.int32),
        ],
        compiler_params=pltpu.CompilerParams(
            dimension_semantics=("arbitrary",)),
    )(x, wt, bb, embed, jnp.arange(N_CODES, dtype=jnp.float32).reshape(N_CODES, 1))

    q_flat, usage3, loss_part = _run_sc(
        idx8.reshape(n * K_TOP), wn.reshape(n * K_TOP),
        z_flat.reshape(n * CODE_DIM), embed.reshape(N_CODES * CODE_DIM), n)

    loss, ent = pl.pallas_call(
        functools.partial(_finalize_kernel, n_rows=n),
        out_shape=[
            jax.ShapeDtypeStruct((1, 1), jnp.float32),
            jax.ShapeDtypeStruct((1, 1), jnp.float32),
        ],
    )(usage3.reshape(NW, N_CODES, 16), loss_part)

    q_st = q_flat.reshape(b, k, CODE_DIM)
    indices = idx.reshape(b, k)
    return (q_st, indices, loss.reshape(()), ent.reshape(()))
